# baseline (speedup 1.0000x reference)
import jax
import jax.numpy as jnp
from jax.experimental import pallas as pl
from jax.experimental.pallas import tpu as pltpu

_M_BLK = 512
_F_SUB = 128


def _ffn_kernel(x_ref, w1_ref, w3_ref, w2_ref, o_ref, xb_ref, inter_ref):
    xb_ref[...] = x_ref[...].astype(jnp.bfloat16)
    xb = xb_ref[...]
    F = w1_ref.shape[1]
    for f in range(F // _F_SUB):
        cols = pl.ds(f * _F_SUB, _F_SUB)
        a = jnp.dot(xb, w1_ref[:, cols], preferred_element_type=jnp.float32)
        b = jnp.dot(xb, w3_ref[:, cols], preferred_element_type=jnp.float32)
        inter_ref[:, cols] = (a * jax.nn.sigmoid(a) * b).astype(jnp.bfloat16)
    o_ref[...] = jnp.dot(
        inter_ref[...], w2_ref[...], preferred_element_type=jnp.float32
    )


def kernel(x, gate_W, W1, W3, W2):
    B, S, H = x.shape
    h = x.reshape(-1, H)
    M = h.shape[0]
    w1 = W1[0].astype(jnp.bfloat16)
    w3 = W3[0].astype(jnp.bfloat16)
    w2 = W2[0].astype(jnp.bfloat16)
    F = w1.shape[1]
    out = pl.pallas_call(
        _ffn_kernel,
        grid=(M // _M_BLK,),
        in_specs=[
            pl.BlockSpec((_M_BLK, H), lambda i: (i, 0)),
            pl.BlockSpec((H, F), lambda i: (0, 0)),
            pl.BlockSpec((H, F), lambda i: (0, 0)),
            pl.BlockSpec((F, H), lambda i: (0, 0)),
        ],
        out_specs=pl.BlockSpec((_M_BLK, H), lambda i: (i, 0)),
        out_shape=jax.ShapeDtypeStruct((M, H), jnp.float32),
        scratch_shapes=[
            pltpu.VMEM((_M_BLK, H), jnp.bfloat16),
            pltpu.VMEM((_M_BLK, F), jnp.bfloat16),
        ],
    )(h, w1, w3, w2)
    return out.reshape(B, S, H)


# speedup vs baseline: 1.5111x; 1.5111x over previous
import jax
import jax.numpy as jnp
from jax.experimental import pallas as pl
from jax.experimental.pallas import tpu as pltpu

_M_BLK = 1024
_F_SUB = 256


def _ffn_kernel(x_ref, w1_ref, w3_ref, w2_ref, o_ref, xb_ref, inter_ref):
    xb_ref[...] = x_ref[...].astype(jnp.bfloat16)
    xb = xb_ref[...]
    F = w1_ref.shape[1]
    for f in range(F // _F_SUB):
        cols = pl.ds(f * _F_SUB, _F_SUB)
        a = jnp.dot(xb, w1_ref[:, cols], preferred_element_type=jnp.float32)
        b = jnp.dot(xb, w3_ref[:, cols], preferred_element_type=jnp.float32)
        inter_ref[:, cols] = (a * jax.nn.sigmoid(a) * b).astype(jnp.bfloat16)
    o_ref[...] = jnp.dot(
        inter_ref[...], w2_ref[...], preferred_element_type=jnp.float32
    )


def kernel(x, gate_W, W1, W3, W2):
    B, S, H = x.shape
    h = x.reshape(-1, H)
    M = h.shape[0]
    w1 = W1[0].astype(jnp.bfloat16)
    w3 = W3[0].astype(jnp.bfloat16)
    w2 = W2[0].astype(jnp.bfloat16)
    F = w1.shape[1]
    out = pl.pallas_call(
        _ffn_kernel,
        grid=(M // _M_BLK,),
        in_specs=[
            pl.BlockSpec((_M_BLK, H), lambda i: (i, 0)),
            pl.BlockSpec((H, F), lambda i: (0, 0)),
            pl.BlockSpec((H, F), lambda i: (0, 0)),
            pl.BlockSpec((F, H), lambda i: (0, 0)),
        ],
        out_specs=pl.BlockSpec((_M_BLK, H), lambda i: (i, 0)),
        out_shape=jax.ShapeDtypeStruct((M, H), jnp.float32),
        scratch_shapes=[
            pltpu.VMEM((_M_BLK, H), jnp.bfloat16),
            pltpu.VMEM((_M_BLK, F), jnp.bfloat16),
        ],
    )(h, w1, w3, w2)
    return out.reshape(B, S, H)


# speedup vs baseline: 1.5134x; 1.0015x over previous
import jax
import jax.numpy as jnp
from jax.experimental import pallas as pl
from jax.experimental.pallas import tpu as pltpu

_M_BLK = 512
_F_SUB = 256
_N_SUB = 512


def _ffn_kernel(x_ref, w1_ref, w3_ref, w2_ref, o_ref, xb_ref, inter_ref):
    xb_ref[...] = x_ref[...].astype(jnp.bfloat16)
    xb = xb_ref[...]
    F = w1_ref.shape[1]
    for f in range(F // _F_SUB):
        cols = pl.ds(f * _F_SUB, _F_SUB)
        a = jnp.dot(xb, w1_ref[:, cols], preferred_element_type=jnp.float32)
        b = jnp.dot(xb, w3_ref[:, cols], preferred_element_type=jnp.float32)
        inter_ref[:, cols] = (a * jax.nn.sigmoid(a) * b).astype(jnp.bfloat16)
    inter = inter_ref[...]
    H = w2_ref.shape[1]
    for n in range(H // _N_SUB):
        ncols = pl.ds(n * _N_SUB, _N_SUB)
        o_ref[:, ncols] = jnp.dot(
            inter, w2_ref[:, ncols], preferred_element_type=jnp.float32
        )


def kernel(x, gate_W, W1, W3, W2):
    B, S, H = x.shape
    h = x.reshape(-1, H)
    M = h.shape[0]
    w1 = W1[0].astype(jnp.bfloat16)
    w3 = W3[0].astype(jnp.bfloat16)
    w2 = W2[0].astype(jnp.bfloat16)
    F = w1.shape[1]
    out = pl.pallas_call(
        _ffn_kernel,
        grid=(M // _M_BLK,),
        in_specs=[
            pl.BlockSpec((_M_BLK, H), lambda i: (i, 0)),
            pl.BlockSpec((H, F), lambda i: (0, 0)),
            pl.BlockSpec((H, F), lambda i: (0, 0)),
            pl.BlockSpec((F, H), lambda i: (0, 0)),
        ],
        out_specs=pl.BlockSpec((_M_BLK, H), lambda i: (i, 0)),
        out_shape=jax.ShapeDtypeStruct((M, H), jnp.float32),
        scratch_shapes=[
            pltpu.VMEM((_M_BLK, H), jnp.bfloat16),
            pltpu.VMEM((_M_BLK, F), jnp.bfloat16),
        ],
    )(h, w1, w3, w2)
    return out.reshape(B, S, H)
